# baseline (speedup 1.0000x reference)
import functools

import jax
import jax.numpy as jnp
from jax import lax
from jax.experimental import pallas as pl
from jax.experimental.pallas import tpu as pltpu
from jax.experimental.pallas import tpu_sc as plsc

N = 1024
E = 65536
SDIM = 256
VDIM = 64
EDIM = 16
NA = 16
NB = 5
G = 32

SPREAD = 16384
NN2 = N * N + SPREAD
DUMP = N * N
ZSPAN = NN2 // 16
EMISS = 4096
NTILE = 16
EPC = E // 2
EPT = EPC // NTILE
EPT_TAB = E // NTILE
CHK = 128


def _dense_body(s_ref, v3_ref, p_ref, b2_ref, wsh_ref, bsh_ref, wct_ref,
                wat_ref, bat_ref, wbd_ref, bbd_ref, wb0_ref, bb0_ref,
                cpc_ref, hw_ref, at_ref, wcomb_ref, cb_ref):
    f32 = jnp.float32
    h = jax.nn.silu(jnp.dot(s_ref[...], wsh_ref[...], preferred_element_type=f32)
                    + bsh_ref[...])
    w0 = wb0_ref[0:SDIM, :]
    hw_ref[...] = jnp.dot(h, w0, preferred_element_type=f32)
    at_ref[...] = jnp.dot(h, wat_ref[...], preferred_element_type=f32) + bat_ref[...]
    t = v3_ref[...] * wct_ref[...]
    r192 = lax.broadcasted_iota(jnp.int32, (3 * VDIM, 16), 0)
    c192 = lax.broadcasted_iota(jnp.int32, (3 * VDIM, 16), 1)
    sel = (r192 // VDIM == c192).astype(f32)
    cp = jnp.dot(t, sel, preferred_element_type=f32)
    r3 = lax.broadcasted_iota(jnp.int32, (3, 16), 0)
    c3 = lax.broadcasted_iota(jnp.int32, (3, 16), 1)
    sel3 = (r3 == c3).astype(f32)
    cp = cp + jnp.dot(p_ref[...], sel3, preferred_element_type=f32)
    gi = lax.broadcasted_iota(jnp.int32, (N, G), 1)
    oh = (b2_ref[...] == gi).astype(f32)
    dn = (((0,), (0,)), ((), ()))
    cnt = lax.dot_general(oh, jnp.ones((N, 1), f32), dn, preferred_element_type=f32)
    sums = lax.dot_general(oh, cp, dn, preferred_element_type=f32)
    mean = sums / jnp.maximum(cnt, 1.0)
    cpc_ref[...] = cp - jnp.dot(oh, mean, preferred_element_type=f32)
    wbw = jnp.dot(wbd_ref[...], w0, preferred_element_type=f32)
    wdrow = wb0_ref[SDIM:SDIM + 1, :]
    wcomb_ref[...] = jnp.concatenate(
        [wbw, wdrow, jnp.zeros((15, SDIM), f32)], axis=0)
    cb_ref[...] = jnp.dot(bbd_ref[...], w0, preferred_element_type=f32) + bb0_ref[...]


def _bond_body(fp_ref, es_ref, wcomb_ref, cb_ref, wb1_ref, bb1_ref, out_ref):
    f32 = jnp.float32
    z = (fp_ref[...]
         + jnp.dot(es_ref[...], wcomb_ref[0:16, :], preferred_element_type=f32)
         + cb_ref[...])
    y = jax.nn.silu(z)
    out_ref[...] = jnp.dot(y, wb1_ref[...], preferred_element_type=f32) + bb1_ref[...]


def _edge_body(j_hbm, i_hbm, ep_hbm, hw_hbm, cp_hbm, wd_hbm,
               fp_hbm, es_hbm, tab_hbm,
               jv, iv, buf_a, buf_b, buf_c, buf_d, dbuf, cp_loc, wd_loc,
               rows_i, rows_j, dirb, revb, esb, sem):
    i32 = jnp.int32
    c = lax.axis_index("c")
    sid = lax.axis_index("s")
    coff = c * NN2
    iota16 = lax.iota(i32, 16)
    zero16 = jnp.zeros((16,), i32)
    one16 = zero16 + 1
    two16 = zero16 + 2
    _s = jax.named_scope("p1_zero")
    _s.__enter__()

    def _zb(t, carry):
        jv[pl.ds(t * 16, 16)] = zero16
        return carry
    lax.fori_loop(0, 256, _zb, None)
    zbase = coff + sid * ZSPAN
    zstarts = [k * 4096 for k in range(ZSPAN // 4096)] + [ZSPAN - 4096]
    hs = [pltpu.async_copy(jv, tab_hbm.at[pl.ds(zbase + st, 4096)], sem)
          for st in zstarts]
    for h in hs:
        h.wait()
    plsc.subcore_barrier()

    _s.__exit__(None, None, None)
    _s = jax.named_scope("p2_tabr1")
    _s.__enter__()
    eb = sid * EPT_TAB
    pltpu.sync_copy(j_hbm.at[pl.ds(eb, EPT_TAB)], jv)
    pltpu.sync_copy(i_hbm.at[pl.ds(eb, EPT_TAB)], iv)

    def _fill1(t, carry):
        m = t // 8
        l = t - m * 8
        sl = pl.ds(t * 16, 16)
        ll = pl.ds(l * 16, 16)
        buf_a[m, ll] = jv[sl] * N + iv[sl] + coff
        buf_b[m, ll] = iota16 + (eb + t * 16 + 1)
        return carry
    lax.fori_loop(0, 256, _fill1, None)
    for g in range(4):
        hs = [pltpu.async_copy(buf_b.at[m], tab_hbm.at[buf_a.at[m]], sem)
              for m in range(g * 8, g * 8 + 8)]
        for h in hs:
            h.wait()

    _s.__exit__(None, None, None)
    _s = jax.named_scope("p3_prep")
    _s.__enter__()
    lb = c * EPC + sid * EPT
    pltpu.sync_copy(j_hbm.at[pl.ds(lb, EPT)], jv.at[pl.ds(0, EPT)])
    pltpu.sync_copy(i_hbm.at[pl.ds(lb, EPT)], iv.at[pl.ds(0, EPT)])
    pltpu.sync_copy(cp_hbm, cp_loc)

    def _fill2(t, carry):
        m = t // 8
        l = t - m * 8
        sl = pl.ds(t * 16, 16)
        ll = pl.ds(l * 16, 16)
        buf_d[m, ll] = iv[sl]
        buf_d[m + 16, ll] = jv[sl]
        return carry
    lax.fori_loop(0, 128, _fill2, None)

    def _dloop(t, carry):
        sl = pl.ds(t * 16, 16)
        ivv = iv[sl] * 16
        jvv = jv[sl] * 16
        xi = plsc.load_gather(cp_loc, [ivv])
        yi = plsc.load_gather(cp_loc, [ivv + one16])
        zi = plsc.load_gather(cp_loc, [ivv + two16])
        xj = plsc.load_gather(cp_loc, [jvv])
        yj = plsc.load_gather(cp_loc, [jvv + one16])
        zj = plsc.load_gather(cp_loc, [jvv + two16])
        dx = xi - xj
        dy = yi - yj
        dz = zi - zj
        dbuf[sl] = dx * dx + dy * dy + dz * dz
        return carry
    lax.fori_loop(0, 128, _dloop, None)

    pltpu.sync_copy(wd_hbm, wd_loc)
    wdsegs = [wd_loc[pl.ds(k * 16, 16)] for k in range(16)]
    _s.__exit__(None, None, None)
    _s = jax.named_scope("p4_fchunk")
    _s.__enter__()

    def _fchunk(ch, carry):
        base = lb + ch * CHK
        h1 = pltpu.async_copy(hw_hbm.at[buf_d.at[ch]], rows_i, sem)
        h2 = pltpu.async_copy(hw_hbm.at[buf_d.at[ch + 16]], rows_j, sem)
        h1.wait()
        h2.wait()

        def _addl(r, cy):
            dv = plsc.load_gather(dbuf, [zero16 + (ch * CHK + r)])
            for k in range(16):
                sl = pl.ds(k * 16, 16)
                rows_i[r, sl] = rows_i[r, sl] + rows_j[r, sl] + dv * wdsegs[k]
            return cy
        lax.fori_loop(0, CHK, _addl, None)
        pltpu.sync_copy(rows_i, fp_hbm.at[pl.ds(base, CHK)])
        return carry
    lax.fori_loop(0, 16, _fchunk, None)

    _s.__exit__(None, None, None)
    _s = jax.named_scope("p5_fix")
    _s.__enter__()
    plsc.subcore_barrier()
    for _ in range(2):
        for g in range(4):
            hs = [pltpu.async_copy(tab_hbm.at[buf_a.at[m]], buf_c.at[m], sem)
                  for m in range(g * 8, g * 8 + 8)]
            for h in hs:
                h.wait()

        def _fix(t, carry):
            m = t // 8
            ll = pl.ds((t - m * 8) * 16, 16)
            wv = buf_c[m, ll]
            idvv = buf_b[m, ll]
            dump = coff + DUMP + (idvv & (SPREAD - 1))
            buf_c[m, ll] = jnp.where(idvv > wv, buf_a[m, ll], dump)
            return carry
        lax.fori_loop(0, 256, _fix, None)
        for g in range(4):
            hs = [pltpu.async_copy(buf_b.at[m], tab_hbm.at[buf_c.at[m]], sem)
                  for m in range(g * 8, g * 8 + 8)]
            for h in hs:
                h.wait()
        plsc.subcore_barrier()

    _s.__exit__(None, None, None)
    _s = jax.named_scope("p6_lookup")
    _s.__enter__()

    def _fillk(t, carry):
        m = t // 8
        sl = pl.ds(t * 16, 16)
        ll = pl.ds((t - m * 8) * 16, 16)
        buf_a[m, ll] = jv[sl] * N + iv[sl] + coff
        buf_a[m + 16, ll] = iv[sl] * N + jv[sl] + coff
        return carry
    lax.fori_loop(0, 128, _fillk, None)
    for g in range(4):
        hs = [pltpu.async_copy(tab_hbm.at[buf_a.at[m]], buf_b.at[m], sem)
              for m in range(g * 8, g * 8 + 8)]
        for h in hs:
            h.wait()

    def _fill3(t, carry):
        m = t // 8
        ll = pl.ds((t - m * 8) * 16, 16)
        buf_c[m, ll] = buf_b[m, ll] - 1
        wrv = buf_b[m + 16, ll]
        miss = E + ((iota16 + (lb + t * 16)) & (EMISS - 1))
        buf_c[m + 16, ll] = jnp.where(wrv == 0, miss, wrv - 1)
        return carry
    lax.fori_loop(0, 128, _fill3, None)
    _s.__exit__(None, None, None)
    _s = jax.named_scope("p7_echunk")
    _s.__enter__()

    def _echunk(ch, carry):
        base = lb + ch * CHK
        h3 = pltpu.async_copy(ep_hbm.at[buf_c.at[ch]], dirb, sem)
        h4 = pltpu.async_copy(ep_hbm.at[buf_c.at[ch + 16]], revb, sem)
        h3.wait()
        h4.wait()

        def _pere(r, cy):
            esb[r, pl.ds(0, 16)] = (dirb[r, pl.ds(0, 16)]
                                    + revb[r, pl.ds(0, 16)]) * 0.5
            return cy
        lax.fori_loop(0, CHK, _pere, None)
        pltpu.sync_copy(esb, es_hbm.at[pl.ds(base, CHK)])
        return carry
    lax.fori_loop(0, 16, _echunk, None)
    _s.__exit__(None, None, None)


def _make_edge_kernel():
    return functools.partial(
        pl.kernel,
        out_type=[
            jax.ShapeDtypeStruct((E, SDIM), jnp.float32),
            jax.ShapeDtypeStruct((E, EDIM), jnp.float32),
            jax.ShapeDtypeStruct((2 * NN2,), jnp.int32),
        ],
        mesh=plsc.VectorSubcoreMesh(core_axis_name="c", subcore_axis_name="s"),
        compiler_params=pltpu.CompilerParams(
            needs_layout_passes=False, use_tc_tiling_on_sc=False),
        scratch_types=[
            pltpu.VMEM((EPT_TAB,), jnp.int32),
            pltpu.VMEM((EPT_TAB,), jnp.int32),
            pltpu.VMEM((32, 128), jnp.int32),
            pltpu.VMEM((32, 128), jnp.int32),
            pltpu.VMEM((32, 128), jnp.int32),
            pltpu.VMEM((32, 128), jnp.int32),
            pltpu.VMEM((EPT,), jnp.float32),
            pltpu.VMEM((N * 16,), jnp.float32),
            pltpu.VMEM((SDIM,), jnp.float32),
            pltpu.VMEM((CHK, SDIM), jnp.float32),
            pltpu.VMEM((CHK, SDIM), jnp.float32),
            pltpu.VMEM((CHK, EDIM), jnp.float32),
            pltpu.VMEM((CHK, EDIM), jnp.float32),
            pltpu.VMEM((CHK, EDIM), jnp.float32),
            pltpu.SemaphoreType.DMA,
        ],
    )(_edge_body)


def kernel(s, v, p, e, batch, edge_index, W_shared, b_shared, W_coords,
           W_atoms, b_atoms, W_bond, b_bond, W_b0, b_b0, W_b1, b_b1):
    f32 = jnp.float32
    b2 = batch.astype(jnp.int32).reshape(N, 1)
    j32 = edge_index[0].astype(jnp.int32)
    i32_ = edge_index[1].astype(jnp.int32)
    v3 = v.reshape(N, 3 * VDIM)
    wct = jnp.tile(W_coords.reshape(-1), 3).reshape(1, 3 * VDIM)
    e_pad = jnp.concatenate([e, jnp.zeros((EMISS, EDIM), f32)], axis=0)

    cpc, hw, atoms, wcomb, cb = pl.pallas_call(
        _dense_body,
        out_shape=[
            jax.ShapeDtypeStruct((N, 16), f32),
            jax.ShapeDtypeStruct((N, SDIM), f32),
            jax.ShapeDtypeStruct((N, NA), f32),
            jax.ShapeDtypeStruct((32, SDIM), f32),
            jax.ShapeDtypeStruct((1, SDIM), f32),
        ],
    )(s, v3, p, b2, W_shared, b_shared.reshape(1, -1), wct,
      W_atoms, b_atoms.reshape(1, -1), W_bond, b_bond.reshape(1, -1),
      W_b0, b_b0.reshape(1, -1))

    fp, es, _tab = _make_edge_kernel()(
        j32, i32_, e_pad, hw, cpc.reshape(-1), W_b0[SDIM])

    ts = 2048
    bonds = pl.pallas_call(
        _bond_body,
        grid=(E // ts,),
        in_specs=[
            pl.BlockSpec((ts, SDIM), lambda i: (i, 0)),
            pl.BlockSpec((ts, EDIM), lambda i: (i, 0)),
            pl.BlockSpec((32, SDIM), lambda i: (0, 0)),
            pl.BlockSpec((1, SDIM), lambda i: (0, 0)),
            pl.BlockSpec((SDIM, NB), lambda i: (0, 0)),
            pl.BlockSpec((1, NB), lambda i: (0, 0)),
        ],
        out_specs=pl.BlockSpec((ts, NB), lambda i: (i, 0)),
        out_shape=jax.ShapeDtypeStruct((E, NB), f32),
    )(fp, es, wcomb, cb, W_b1, b_b1.reshape(1, -1))

    return (cpc[:, :3], atoms, bonds)


# speedup vs baseline: 1.5878x; 1.5878x over previous
import functools

import jax
import jax.numpy as jnp
from jax import lax
from jax.experimental import pallas as pl
from jax.experimental.pallas import tpu as pltpu
from jax.experimental.pallas import tpu_sc as plsc

N = 1024
E = 65536
SDIM = 256
VDIM = 64
EDIM = 16
NA = 16
NB = 5
G = 32

NN2 = 2 * N * N
DUMP = N * N
ZSPAN = N * N // 16
EMISS = 4096
NTILE = 16
EPC = E // 2
EPT = EPC // NTILE
EPT_TAB = E // NTILE
CHK = 128


def _dense_body(s_ref, v3_ref, p_ref, b2_ref, wsh_ref, bsh_ref, wct_ref,
                wat_ref, bat_ref, wbd_ref, bbd_ref, wb0_ref, bb0_ref,
                cpc_ref, hw_ref, at_ref, wcomb_ref, cb_ref):
    f32 = jnp.float32
    h = jax.nn.silu(jnp.dot(s_ref[...], wsh_ref[...], preferred_element_type=f32)
                    + bsh_ref[...])
    w0 = wb0_ref[0:SDIM, :]
    hw_ref[...] = jnp.dot(h, w0, preferred_element_type=f32)
    at_ref[...] = jnp.dot(h, wat_ref[...], preferred_element_type=f32) + bat_ref[...]
    t = v3_ref[...] * wct_ref[...]
    r192 = lax.broadcasted_iota(jnp.int32, (3 * VDIM, 16), 0)
    c192 = lax.broadcasted_iota(jnp.int32, (3 * VDIM, 16), 1)
    sel = (r192 // VDIM == c192).astype(f32)
    cp = jnp.dot(t, sel, preferred_element_type=f32)
    r3 = lax.broadcasted_iota(jnp.int32, (3, 16), 0)
    c3 = lax.broadcasted_iota(jnp.int32, (3, 16), 1)
    sel3 = (r3 == c3).astype(f32)
    cp = cp + jnp.dot(p_ref[...], sel3, preferred_element_type=f32)
    gi = lax.broadcasted_iota(jnp.int32, (N, G), 1)
    oh = (b2_ref[...] == gi).astype(f32)
    dn = (((0,), (0,)), ((), ()))
    cnt = lax.dot_general(oh, jnp.ones((N, 1), f32), dn, preferred_element_type=f32)
    sums = lax.dot_general(oh, cp, dn, preferred_element_type=f32)
    mean = sums / jnp.maximum(cnt, 1.0)
    cpc_ref[...] = cp - jnp.dot(oh, mean, preferred_element_type=f32)
    wbw = jnp.dot(wbd_ref[...], w0, preferred_element_type=f32)
    wdrow = wb0_ref[SDIM:SDIM + 1, :]
    wcomb_ref[...] = jnp.concatenate(
        [wbw, wdrow, jnp.zeros((15, SDIM), f32)], axis=0)
    cb_ref[...] = jnp.dot(bbd_ref[...], w0, preferred_element_type=f32) + bb0_ref[...]


def _bond_body(fp_ref, es_ref, wcomb_ref, cb_ref, wb1_ref, bb1_ref, out_ref):
    f32 = jnp.float32
    z = (fp_ref[...]
         + jnp.dot(es_ref[...], wcomb_ref[0:16, :], preferred_element_type=f32)
         + cb_ref[...])
    y = jax.nn.silu(z)
    out_ref[...] = jnp.dot(y, wb1_ref[...], preferred_element_type=f32) + bb1_ref[...]


def _edge_body(j_hbm, i_hbm, ep_hbm, hw_hbm, cp_hbm, wd_hbm,
               fp_hbm, es_hbm, tab_hbm,
               jv, iv, buf_a, buf_b, buf_c, buf_d, dbuf, cp_loc, wd_loc,
               a_i, a_j, b_i, b_j, dirb, revb, dirb2, revb2, esb, esb2,
               sem, sem2):
    i32 = jnp.int32
    c = lax.axis_index("c")
    sid = lax.axis_index("s")
    coff = c * NN2
    iota16 = lax.iota(i32, 16)
    zero16 = jnp.zeros((16,), i32)
    one16 = zero16 + 1
    two16 = zero16 + 2
    _s = jax.named_scope("p1_zero")
    _s.__enter__()

    def _zb(t, carry):
        jv[pl.ds(t * 16, 16)] = zero16
        return carry
    lax.fori_loop(0, 256, _zb, None)
    zbase = coff + sid * ZSPAN
    zstarts = [k * 4096 for k in range(ZSPAN // 4096)]
    hs = [pltpu.async_copy(jv, tab_hbm.at[pl.ds(zbase + st, 4096)], sem)
          for st in zstarts]
    for h in hs:
        h.wait()
    plsc.subcore_barrier()

    _s.__exit__(None, None, None)
    _s = jax.named_scope("p2_tabr1")
    _s.__enter__()
    eb = sid * EPT_TAB
    pltpu.sync_copy(j_hbm.at[pl.ds(eb, EPT_TAB)], jv)
    pltpu.sync_copy(i_hbm.at[pl.ds(eb, EPT_TAB)], iv)

    def _fill1(t, carry):
        m = t // 8
        l = t - m * 8
        sl = pl.ds(t * 16, 16)
        ll = pl.ds(l * 16, 16)
        buf_a[m, ll] = jv[sl] * N + iv[sl] + coff
        buf_b[m, ll] = iota16 + (eb + t * 16 + 1)
        return carry
    lax.fori_loop(0, 256, _fill1, None)
    for g in range(4):
        hs = [pltpu.async_copy(buf_b.at[m], tab_hbm.at[buf_a.at[m]], sem)
              for m in range(g * 8, g * 8 + 8)]
        for h in hs:
            h.wait()

    _s.__exit__(None, None, None)
    _s = jax.named_scope("p3_prep")
    _s.__enter__()
    lb = c * EPC + sid * EPT
    pltpu.sync_copy(j_hbm.at[pl.ds(lb, EPT)], jv.at[pl.ds(0, EPT)])
    pltpu.sync_copy(i_hbm.at[pl.ds(lb, EPT)], iv.at[pl.ds(0, EPT)])
    pltpu.sync_copy(cp_hbm, cp_loc)

    def _fill2(t, carry):
        m = t // 8
        l = t - m * 8
        sl = pl.ds(t * 16, 16)
        ll = pl.ds(l * 16, 16)
        buf_d[m, ll] = iv[sl]
        buf_d[m + 16, ll] = jv[sl]
        return carry
    lax.fori_loop(0, 128, _fill2, None)

    def _dloop(t, carry):
        sl = pl.ds(t * 16, 16)
        ivv = iv[sl] * 16
        jvv = jv[sl] * 16
        xi = plsc.load_gather(cp_loc, [ivv])
        yi = plsc.load_gather(cp_loc, [ivv + one16])
        zi = plsc.load_gather(cp_loc, [ivv + two16])
        xj = plsc.load_gather(cp_loc, [jvv])
        yj = plsc.load_gather(cp_loc, [jvv + one16])
        zj = plsc.load_gather(cp_loc, [jvv + two16])
        dx = xi - xj
        dy = yi - yj
        dz = zi - zj
        dbuf[sl] = dx * dx + dy * dy + dz * dz
        return carry
    lax.fori_loop(0, 128, _dloop, None)

    pltpu.sync_copy(wd_hbm, wd_loc)
    wdsegs = [wd_loc[pl.ds(k * 16, 16)] for k in range(16)]
    _s.__exit__(None, None, None)
    _s = jax.named_scope("p4_fchunk")
    _s.__enter__()

    def _fidx(base_row, ch):
        return buf_d.at[base_row + ch // 2, pl.ds((ch % 2) * 64, 64)]

    def _ffire(ch, bi, bj, s):
        pltpu.async_copy(hw_hbm.at[_fidx(0, ch)], bi, s)
        pltpu.async_copy(hw_hbm.at[_fidx(16, ch)], bj, s)

    def _fdrain(bi, bj, s):
        pltpu.make_async_copy(hw_hbm.at[_fidx(0, 0)], bi, s).wait()
        pltpu.make_async_copy(hw_hbm.at[_fidx(16, 0)], bj, s).wait()

    def _fproc(ch, bi, bj):
        def _addl(r, cy):
            dv = plsc.load_gather(dbuf, [zero16 + (ch * 64 + r)])
            for k in range(16):
                sl = pl.ds(k * 16, 16)
                bi[r, sl] = bi[r, sl] + bj[r, sl] + dv * wdsegs[k]
            return cy
        lax.fori_loop(0, 64, _addl, None)
        pltpu.sync_copy(bi, fp_hbm.at[pl.ds(lb + ch * 64, 64)])

    _ffire(0, a_i, a_j, sem)

    def _fpipe(k, carry):
        c1 = 2 * k + 1
        _ffire(c1, b_i, b_j, sem2)
        _fdrain(a_i, a_j, sem)
        _fproc(2 * k, a_i, a_j)
        _ffire(jnp.minimum(2 * k + 2, 31), a_i, a_j, sem)
        _fdrain(b_i, b_j, sem2)
        _fproc(c1, b_i, b_j)
        return carry
    lax.fori_loop(0, 16, _fpipe, None)
    _fdrain(a_i, a_j, sem)

    _s.__exit__(None, None, None)
    _s = jax.named_scope("p5_fix")
    _s.__enter__()
    plsc.subcore_barrier()
    for _ in range(2):
        for g in range(4):
            hs = [pltpu.async_copy(tab_hbm.at[buf_a.at[m]], buf_c.at[m], sem)
                  for m in range(g * 8, g * 8 + 8)]
            for h in hs:
                h.wait()

        def _fix(t, carry):
            m = t // 8
            ll = pl.ds((t - m * 8) * 16, 16)
            wv = buf_c[m, ll]
            idvv = buf_b[m, ll]
            dump = coff + DUMP + ((idvv & (E - 1)) * 16)
            buf_c[m, ll] = jnp.where(idvv > wv, buf_a[m, ll], dump)
            return carry
        lax.fori_loop(0, 256, _fix, None)
        for g in range(4):
            hs = [pltpu.async_copy(buf_b.at[m], tab_hbm.at[buf_c.at[m]], sem)
                  for m in range(g * 8, g * 8 + 8)]
            for h in hs:
                h.wait()
        plsc.subcore_barrier()

    _s.__exit__(None, None, None)
    _s = jax.named_scope("p6_lookup")
    _s.__enter__()

    def _fillk(t, carry):
        m = t // 8
        sl = pl.ds(t * 16, 16)
        ll = pl.ds((t - m * 8) * 16, 16)
        buf_a[m, ll] = jv[sl] * N + iv[sl] + coff
        buf_a[m + 16, ll] = iv[sl] * N + jv[sl] + coff
        return carry
    lax.fori_loop(0, 128, _fillk, None)
    for g in range(4):
        hs = [pltpu.async_copy(tab_hbm.at[buf_a.at[m]], buf_b.at[m], sem)
              for m in range(g * 8, g * 8 + 8)]
        for h in hs:
            h.wait()

    def _fill3(t, carry):
        m = t // 8
        ll = pl.ds((t - m * 8) * 16, 16)
        buf_c[m, ll] = buf_b[m, ll] - 1
        wrv = buf_b[m + 16, ll]
        miss = E + ((iota16 + (lb + t * 16)) & (EMISS - 1))
        buf_c[m + 16, ll] = jnp.where(wrv == 0, miss, wrv - 1)
        return carry
    lax.fori_loop(0, 128, _fill3, None)
    _s.__exit__(None, None, None)
    _s = jax.named_scope("p7_echunk")
    _s.__enter__()

    def _eidx(base_row, ch):
        return buf_c.at[base_row + ch // 2, pl.ds((ch % 2) * 64, 64)]

    def _efire(ch, bd, br, s):
        pltpu.async_copy(ep_hbm.at[_eidx(0, ch)], bd, s)
        pltpu.async_copy(ep_hbm.at[_eidx(16, ch)], br, s)

    def _edrain(bd, br, s):
        pltpu.make_async_copy(ep_hbm.at[_eidx(0, 0)], bd, s).wait()
        pltpu.make_async_copy(ep_hbm.at[_eidx(16, 0)], br, s).wait()

    def _eproc(ch, bd, br, bo):
        def _pere(r, cy):
            bo[r, pl.ds(0, 16)] = (bd[r, pl.ds(0, 16)]
                                   + br[r, pl.ds(0, 16)]) * 0.5
            return cy
        lax.fori_loop(0, 64, _pere, None)
        pltpu.sync_copy(bo, es_hbm.at[pl.ds(lb + ch * 64, 64)])

    _efire(0, dirb, revb, sem)

    def _epipe(k, carry):
        c1 = 2 * k + 1
        _efire(c1, dirb2, revb2, sem2)
        _edrain(dirb, revb, sem)
        _eproc(2 * k, dirb, revb, esb)
        _efire(jnp.minimum(2 * k + 2, 31), dirb, revb, sem)
        _edrain(dirb2, revb2, sem2)
        _eproc(c1, dirb2, revb2, esb2)
        return carry
    lax.fori_loop(0, 16, _epipe, None)
    _edrain(dirb, revb, sem)
    _s.__exit__(None, None, None)


def _make_edge_kernel():
    return functools.partial(
        pl.kernel,
        out_type=[
            jax.ShapeDtypeStruct((E, SDIM), jnp.float32),
            jax.ShapeDtypeStruct((E, EDIM), jnp.float32),
            jax.ShapeDtypeStruct((2 * NN2,), jnp.int32),
        ],
        mesh=plsc.VectorSubcoreMesh(core_axis_name="c", subcore_axis_name="s"),
        compiler_params=pltpu.CompilerParams(
            needs_layout_passes=False, use_tc_tiling_on_sc=False),
        scratch_types=[
            pltpu.VMEM((EPT_TAB,), jnp.int32),
            pltpu.VMEM((EPT_TAB,), jnp.int32),
            pltpu.VMEM((32, 128), jnp.int32),
            pltpu.VMEM((32, 128), jnp.int32),
            pltpu.VMEM((32, 128), jnp.int32),
            pltpu.VMEM((32, 128), jnp.int32),
            pltpu.VMEM((EPT,), jnp.float32),
            pltpu.VMEM((N * 16,), jnp.float32),
            pltpu.VMEM((SDIM,), jnp.float32),
            pltpu.VMEM((64, SDIM), jnp.float32),
            pltpu.VMEM((64, SDIM), jnp.float32),
            pltpu.VMEM((64, SDIM), jnp.float32),
            pltpu.VMEM((64, SDIM), jnp.float32),
            pltpu.VMEM((64, EDIM), jnp.float32),
            pltpu.VMEM((64, EDIM), jnp.float32),
            pltpu.VMEM((64, EDIM), jnp.float32),
            pltpu.VMEM((64, EDIM), jnp.float32),
            pltpu.VMEM((64, EDIM), jnp.float32),
            pltpu.VMEM((64, EDIM), jnp.float32),
            pltpu.SemaphoreType.DMA,
            pltpu.SemaphoreType.DMA,
        ],
    )(_edge_body)


def kernel(s, v, p, e, batch, edge_index, W_shared, b_shared, W_coords,
           W_atoms, b_atoms, W_bond, b_bond, W_b0, b_b0, W_b1, b_b1):
    f32 = jnp.float32
    b2 = batch.astype(jnp.int32).reshape(N, 1)
    j32 = edge_index[0].astype(jnp.int32)
    i32_ = edge_index[1].astype(jnp.int32)
    v3 = v.reshape(N, 3 * VDIM)
    wct = jnp.tile(W_coords.reshape(-1), 3).reshape(1, 3 * VDIM)
    e_pad = jnp.concatenate([e, jnp.zeros((EMISS, EDIM), f32)], axis=0)

    cpc, hw, atoms, wcomb, cb = pl.pallas_call(
        _dense_body,
        out_shape=[
            jax.ShapeDtypeStruct((N, 16), f32),
            jax.ShapeDtypeStruct((N, SDIM), f32),
            jax.ShapeDtypeStruct((N, NA), f32),
            jax.ShapeDtypeStruct((32, SDIM), f32),
            jax.ShapeDtypeStruct((1, SDIM), f32),
        ],
    )(s, v3, p, b2, W_shared, b_shared.reshape(1, -1), wct,
      W_atoms, b_atoms.reshape(1, -1), W_bond, b_bond.reshape(1, -1),
      W_b0, b_b0.reshape(1, -1))

    fp, es, _tab = _make_edge_kernel()(
        j32, i32_, e_pad, hw, cpc.reshape(-1), W_b0[SDIM])

    ts = 2048
    bonds = pl.pallas_call(
        _bond_body,
        grid=(E // ts,),
        in_specs=[
            pl.BlockSpec((ts, SDIM), lambda i: (i, 0)),
            pl.BlockSpec((ts, EDIM), lambda i: (i, 0)),
            pl.BlockSpec((32, SDIM), lambda i: (0, 0)),
            pl.BlockSpec((1, SDIM), lambda i: (0, 0)),
            pl.BlockSpec((SDIM, NB), lambda i: (0, 0)),
            pl.BlockSpec((1, NB), lambda i: (0, 0)),
        ],
        out_specs=pl.BlockSpec((ts, NB), lambda i: (i, 0)),
        out_shape=jax.ShapeDtypeStruct((E, NB), f32),
    )(fp, es, wcomb, cb, W_b1, b_b1.reshape(1, -1))

    return (cpc[:, :3], atoms, bonds)


# speedup vs baseline: 1.8704x; 1.1780x over previous
import functools

import jax
import jax.numpy as jnp
from jax import lax
from jax.experimental import pallas as pl
from jax.experimental.pallas import tpu as pltpu
from jax.experimental.pallas import tpu_sc as plsc

N = 1024
E = 65536
SDIM = 256
VDIM = 64
EDIM = 16
NA = 16
NB = 5
G = 32

NN2 = 2 * N * N
DUMP = N * N
ZSPAN = N * N // 16
EMISS = 4096
NTILE = 16
EPC = E // 2
EPT = EPC // NTILE
EPT_TAB = E // NTILE
CHK = 128


def _dense_body(s_ref, v3_ref, p_ref, b2_ref, wsh_ref, bsh_ref, wct_ref,
                wat_ref, bat_ref, wbd_ref, bbd_ref, wb0_ref, bb0_ref,
                cpc_ref, hw_ref, at_ref, wcomb_ref, cb_ref):
    f32 = jnp.float32
    h = jax.nn.silu(jnp.dot(s_ref[...], wsh_ref[...], preferred_element_type=f32)
                    + bsh_ref[...])
    w0 = wb0_ref[0:SDIM, :]
    hw_ref[...] = jnp.dot(h, w0, preferred_element_type=f32)
    at_ref[...] = jnp.dot(h, wat_ref[...], preferred_element_type=f32) + bat_ref[...]
    t = v3_ref[...] * wct_ref[...]
    r192 = lax.broadcasted_iota(jnp.int32, (3 * VDIM, 16), 0)
    c192 = lax.broadcasted_iota(jnp.int32, (3 * VDIM, 16), 1)
    sel = (r192 // VDIM == c192).astype(f32)
    cp = jnp.dot(t, sel, preferred_element_type=f32)
    r3 = lax.broadcasted_iota(jnp.int32, (3, 16), 0)
    c3 = lax.broadcasted_iota(jnp.int32, (3, 16), 1)
    sel3 = (r3 == c3).astype(f32)
    cp = cp + jnp.dot(p_ref[...], sel3, preferred_element_type=f32)
    gi = lax.broadcasted_iota(jnp.int32, (N, G), 1)
    oh = (b2_ref[...] == gi).astype(f32)
    dn = (((0,), (0,)), ((), ()))
    cnt = lax.dot_general(oh, jnp.ones((N, 1), f32), dn, preferred_element_type=f32)
    sums = lax.dot_general(oh, cp, dn, preferred_element_type=f32)
    mean = sums / jnp.maximum(cnt, 1.0)
    cpc_ref[...] = cp - jnp.dot(oh, mean, preferred_element_type=f32)
    wbw = jnp.dot(wbd_ref[...], w0, preferred_element_type=f32)
    wdrow = wb0_ref[SDIM:SDIM + 1, :]
    wcomb_ref[...] = jnp.concatenate(
        [wbw, wdrow, jnp.zeros((15, SDIM), f32)], axis=0)
    cb_ref[...] = jnp.dot(bbd_ref[...], w0, preferred_element_type=f32) + bb0_ref[...]


def _bond_body(fp_ref, es_ref, wcomb_ref, cb_ref, wb1_ref, bb1_ref, out_ref):
    f32 = jnp.float32
    z = (fp_ref[...]
         + jnp.dot(es_ref[...], wcomb_ref[0:16, :], preferred_element_type=f32)
         + cb_ref[...])
    y = jax.nn.silu(z)
    out_ref[...] = jnp.dot(y, wb1_ref[...], preferred_element_type=f32) + bb1_ref[...]


def _edge_body(j_hbm, i_hbm, ep_hbm, hw_hbm, cp_hbm, wd_hbm,
               fp_hbm, es_hbm, tab_hbm,
               jv, iv, buf_a, buf_b, buf_c, buf_d, dbuf, cp_loc, wd_loc,
               a_i, a_j, b_i, b_j, dirb, revb, dirb2, revb2, esb, esb2,
               sem, sem2, sem3):
    i32 = jnp.int32
    c = lax.axis_index("c")
    sid = lax.axis_index("s")
    coff = c * NN2
    iota16 = lax.iota(i32, 16)
    zero16 = jnp.zeros((16,), i32)
    one16 = zero16 + 1
    two16 = zero16 + 2
    _s = jax.named_scope("p1_zero")
    _s.__enter__()

    def _zb(t, carry):
        jv[pl.ds(t * 16, 16)] = zero16
        return carry
    lax.fori_loop(0, 256, _zb, None)
    zbase = coff + sid * ZSPAN
    zstarts = [k * 4096 for k in range(ZSPAN // 4096)]
    hs = [pltpu.async_copy(jv, tab_hbm.at[pl.ds(zbase + st, 4096)], sem)
          for st in zstarts]
    for h in hs:
        h.wait()
    plsc.subcore_barrier()

    _s.__exit__(None, None, None)
    _s = jax.named_scope("p2_tabr1")
    _s.__enter__()
    eb = sid * EPT_TAB
    pltpu.sync_copy(j_hbm.at[pl.ds(eb, EPT_TAB)], jv)
    pltpu.sync_copy(i_hbm.at[pl.ds(eb, EPT_TAB)], iv)

    def _fill1(t, carry):
        m = t // 8
        l = t - m * 8
        sl = pl.ds(t * 16, 16)
        ll = pl.ds(l * 16, 16)
        buf_a[m, ll] = jv[sl] * N + iv[sl] + coff
        buf_b[m, ll] = iota16 + (eb + t * 16 + 1)
        return carry
    lax.fori_loop(0, 256, _fill1, None)
    for m in range(32):
        pltpu.async_copy(buf_b.at[m], tab_hbm.at[buf_a.at[m]], sem3)

    _s.__exit__(None, None, None)
    _s = jax.named_scope("p3_prep")
    _s.__enter__()
    lb = c * EPC + sid * EPT
    pltpu.sync_copy(j_hbm.at[pl.ds(lb, EPT)], jv.at[pl.ds(0, EPT)])
    pltpu.sync_copy(i_hbm.at[pl.ds(lb, EPT)], iv.at[pl.ds(0, EPT)])
    pltpu.sync_copy(cp_hbm, cp_loc)

    def _fill2(t, carry):
        m = t // 8
        l = t - m * 8
        sl = pl.ds(t * 16, 16)
        ll = pl.ds(l * 16, 16)
        buf_d[m, ll] = iv[sl]
        buf_d[m + 16, ll] = jv[sl]
        return carry
    lax.fori_loop(0, 128, _fill2, None)

    def _dloop(t, carry):
        sl = pl.ds(t * 16, 16)
        ivv = iv[sl] * 16
        jvv = jv[sl] * 16
        xi = plsc.load_gather(cp_loc, [ivv])
        yi = plsc.load_gather(cp_loc, [ivv + one16])
        zi = plsc.load_gather(cp_loc, [ivv + two16])
        xj = plsc.load_gather(cp_loc, [jvv])
        yj = plsc.load_gather(cp_loc, [jvv + one16])
        zj = plsc.load_gather(cp_loc, [jvv + two16])
        dx = xi - xj
        dy = yi - yj
        dz = zi - zj
        dbuf[sl] = dx * dx + dy * dy + dz * dz
        return carry
    lax.fori_loop(0, 128, _dloop, None)

    pltpu.sync_copy(wd_hbm, wd_loc)
    wdsegs = [wd_loc[pl.ds(k * 16, 16)] for k in range(16)]
    _s.__exit__(None, None, None)
    _s = jax.named_scope("p4_fchunk")
    _s.__enter__()

    def _fidx(base_row, ch):
        return buf_d.at[base_row + ch // 2, pl.ds((ch % 2) * 64, 64)]

    def _ffire(ch, bi, bj, s):
        pltpu.async_copy(hw_hbm.at[_fidx(0, ch)], bi, s)
        pltpu.async_copy(hw_hbm.at[_fidx(16, ch)], bj, s)

    def _fdrain(bi, bj, s):
        pltpu.make_async_copy(hw_hbm.at[_fidx(0, 0)], bi, s).wait()
        pltpu.make_async_copy(hw_hbm.at[_fidx(16, 0)], bj, s).wait()

    def _fproc(ch, bi, bj):
        def _addl(r, cy):
            dv = plsc.load_gather(dbuf, [zero16 + (ch * 64 + r)])
            for k in range(16):
                sl = pl.ds(k * 16, 16)
                bi[r, sl] = bi[r, sl] + bj[r, sl] + dv * wdsegs[k]
            return cy
        lax.fori_loop(0, 64, _addl, None)
        pltpu.sync_copy(bi, fp_hbm.at[pl.ds(lb + ch * 64, 64)])

    _ffire(0, a_i, a_j, sem)

    def _fpipe(k, carry):
        c1 = 2 * k + 1
        _ffire(c1, b_i, b_j, sem2)
        _fdrain(a_i, a_j, sem)
        _fproc(2 * k, a_i, a_j)
        _ffire(jnp.minimum(2 * k + 2, 31), a_i, a_j, sem)
        _fdrain(b_i, b_j, sem2)
        _fproc(c1, b_i, b_j)
        return carry
    lax.fori_loop(0, 16, _fpipe, None)
    _fdrain(a_i, a_j, sem)

    _s.__exit__(None, None, None)
    _s = jax.named_scope("p5_fix")
    _s.__enter__()
    for m in range(32):
        pltpu.make_async_copy(buf_b.at[m], tab_hbm.at[buf_a.at[m]], sem3).wait()
    plsc.subcore_barrier()
    for _ in range(2):
        for g in range(4):
            hs = [pltpu.async_copy(tab_hbm.at[buf_a.at[m]], buf_c.at[m], sem)
                  for m in range(g * 8, g * 8 + 8)]
            for h in hs:
                h.wait()

        def _fix(t, carry):
            m = t // 8
            ll = pl.ds((t - m * 8) * 16, 16)
            wv = buf_c[m, ll]
            idvv = buf_b[m, ll]
            dump = coff + DUMP + ((idvv * 1103515245) & (N * N - 1))
            buf_c[m, ll] = jnp.where(idvv > wv, buf_a[m, ll], dump)
            return carry
        lax.fori_loop(0, 256, _fix, None)
        for g in range(4):
            hs = [pltpu.async_copy(buf_b.at[m], tab_hbm.at[buf_c.at[m]], sem)
                  for m in range(g * 8, g * 8 + 8)]
            for h in hs:
                h.wait()
        plsc.subcore_barrier()

    _s.__exit__(None, None, None)
    _s = jax.named_scope("p6_lookup")
    _s.__enter__()

    def _fillk(t, carry):
        m = t // 8
        sl = pl.ds(t * 16, 16)
        ll = pl.ds((t - m * 8) * 16, 16)
        buf_a[m, ll] = jv[sl] * N + iv[sl] + coff
        buf_a[m + 16, ll] = iv[sl] * N + jv[sl] + coff
        return carry
    lax.fori_loop(0, 128, _fillk, None)
    for g in range(4):
        hs = [pltpu.async_copy(tab_hbm.at[buf_a.at[m]], buf_b.at[m], sem)
              for m in range(g * 8, g * 8 + 8)]
        for h in hs:
            h.wait()

    def _fill3(t, carry):
        m = t // 8
        ll = pl.ds((t - m * 8) * 16, 16)
        buf_c[m, ll] = buf_b[m, ll] - 1
        wrv = buf_b[m + 16, ll]
        miss = E + ((iota16 + (lb + t * 16)) & (EMISS - 1))
        buf_c[m + 16, ll] = jnp.where(wrv == 0, miss, wrv - 1)
        return carry
    lax.fori_loop(0, 128, _fill3, None)
    _s.__exit__(None, None, None)
    _s = jax.named_scope("p7_echunk")
    _s.__enter__()

    def _eidx(base_row, ch):
        return buf_c.at[base_row + ch // 2, pl.ds((ch % 2) * 64, 64)]

    def _efire(ch, bd, br, s):
        pltpu.async_copy(ep_hbm.at[_eidx(0, ch)], bd, s)
        pltpu.async_copy(ep_hbm.at[_eidx(16, ch)], br, s)

    def _edrain(bd, br, s):
        pltpu.make_async_copy(ep_hbm.at[_eidx(0, 0)], bd, s).wait()
        pltpu.make_async_copy(ep_hbm.at[_eidx(16, 0)], br, s).wait()

    def _eproc(ch, bd, br, bo):
        def _pere(r, cy):
            bo[r, pl.ds(0, 16)] = (bd[r, pl.ds(0, 16)]
                                   + br[r, pl.ds(0, 16)]) * 0.5
            return cy
        lax.fori_loop(0, 64, _pere, None)
        pltpu.sync_copy(bo, es_hbm.at[pl.ds(lb + ch * 64, 64)])

    _efire(0, dirb, revb, sem)

    def _epipe(k, carry):
        c1 = 2 * k + 1
        _efire(c1, dirb2, revb2, sem2)
        _edrain(dirb, revb, sem)
        _eproc(2 * k, dirb, revb, esb)
        _efire(jnp.minimum(2 * k + 2, 31), dirb, revb, sem)
        _edrain(dirb2, revb2, sem2)
        _eproc(c1, dirb2, revb2, esb2)
        return carry
    lax.fori_loop(0, 16, _epipe, None)
    _edrain(dirb, revb, sem)
    _s.__exit__(None, None, None)


def _make_edge_kernel():
    return functools.partial(
        pl.kernel,
        out_type=[
            jax.ShapeDtypeStruct((E, SDIM), jnp.float32),
            jax.ShapeDtypeStruct((E, EDIM), jnp.float32),
            jax.ShapeDtypeStruct((2 * NN2,), jnp.int32),
        ],
        mesh=plsc.VectorSubcoreMesh(core_axis_name="c", subcore_axis_name="s"),
        compiler_params=pltpu.CompilerParams(
            needs_layout_passes=False, use_tc_tiling_on_sc=False),
        scratch_types=[
            pltpu.VMEM((EPT_TAB,), jnp.int32),
            pltpu.VMEM((EPT_TAB,), jnp.int32),
            pltpu.VMEM((32, 128), jnp.int32),
            pltpu.VMEM((32, 128), jnp.int32),
            pltpu.VMEM((32, 128), jnp.int32),
            pltpu.VMEM((32, 128), jnp.int32),
            pltpu.VMEM((EPT,), jnp.float32),
            pltpu.VMEM((N * 16,), jnp.float32),
            pltpu.VMEM((SDIM,), jnp.float32),
            pltpu.VMEM((64, SDIM), jnp.float32),
            pltpu.VMEM((64, SDIM), jnp.float32),
            pltpu.VMEM((64, SDIM), jnp.float32),
            pltpu.VMEM((64, SDIM), jnp.float32),
            pltpu.VMEM((64, EDIM), jnp.float32),
            pltpu.VMEM((64, EDIM), jnp.float32),
            pltpu.VMEM((64, EDIM), jnp.float32),
            pltpu.VMEM((64, EDIM), jnp.float32),
            pltpu.VMEM((64, EDIM), jnp.float32),
            pltpu.VMEM((64, EDIM), jnp.float32),
            pltpu.SemaphoreType.DMA,
            pltpu.SemaphoreType.DMA,
            pltpu.SemaphoreType.DMA,
        ],
    )(_edge_body)


def kernel(s, v, p, e, batch, edge_index, W_shared, b_shared, W_coords,
           W_atoms, b_atoms, W_bond, b_bond, W_b0, b_b0, W_b1, b_b1):
    f32 = jnp.float32
    b2 = batch.astype(jnp.int32).reshape(N, 1)
    j32 = edge_index[0].astype(jnp.int32)
    i32_ = edge_index[1].astype(jnp.int32)
    v3 = v.reshape(N, 3 * VDIM)
    wct = jnp.tile(W_coords.reshape(-1), 3).reshape(1, 3 * VDIM)
    e_pad = jnp.concatenate([e, jnp.zeros((EMISS, EDIM), f32)], axis=0)

    cpc, hw, atoms, wcomb, cb = pl.pallas_call(
        _dense_body,
        out_shape=[
            jax.ShapeDtypeStruct((N, 16), f32),
            jax.ShapeDtypeStruct((N, SDIM), f32),
            jax.ShapeDtypeStruct((N, NA), f32),
            jax.ShapeDtypeStruct((32, SDIM), f32),
            jax.ShapeDtypeStruct((1, SDIM), f32),
        ],
    )(s, v3, p, b2, W_shared, b_shared.reshape(1, -1), wct,
      W_atoms, b_atoms.reshape(1, -1), W_bond, b_bond.reshape(1, -1),
      W_b0, b_b0.reshape(1, -1))

    fp, es, _tab = _make_edge_kernel()(
        j32, i32_, e_pad, hw, cpc.reshape(-1), W_b0[SDIM])

    ts = 2048
    bonds = pl.pallas_call(
        _bond_body,
        grid=(E // ts,),
        in_specs=[
            pl.BlockSpec((ts, SDIM), lambda i: (i, 0)),
            pl.BlockSpec((ts, EDIM), lambda i: (i, 0)),
            pl.BlockSpec((32, SDIM), lambda i: (0, 0)),
            pl.BlockSpec((1, SDIM), lambda i: (0, 0)),
            pl.BlockSpec((SDIM, NB), lambda i: (0, 0)),
            pl.BlockSpec((1, NB), lambda i: (0, 0)),
        ],
        out_specs=pl.BlockSpec((ts, NB), lambda i: (i, 0)),
        out_shape=jax.ShapeDtypeStruct((E, NB), f32),
    )(fp, es, wcomb, cb, W_b1, b_b1.reshape(1, -1))

    return (cpc[:, :3], atoms, bonds)
